# Initial kernel scaffold; baseline (speedup 1.0000x reference)
import jax
import jax.numpy as jnp
from jax.experimental import pallas as pl


def kernel(input, W1, b1, W2, b2, W3, b3, L1, lb1, C1, cb1, L2, lb2, C2, cb2, M1, mb1, M2, mb2, F11, fb11, F12, fb12, F13, fb13, F21, fb21, F22, fb22, F23, fb23):
    raise NotImplementedError("write your pallas kernel here")


# speedup vs baseline: 15.4236x; 15.4236x over previous
import functools

import jax
import jax.numpy as jnp
from jax.experimental import pallas as pl

_INTERPRET = False

K_NN = 16
M_PTS = 2048
F32 = jnp.float32
BF = jnp.bfloat16


def _relu(x):
    return jnp.maximum(x, 0.0)


def _dot(a, b):
    return jax.lax.dot(a, b, preferred_element_type=F32)


def _bdot(a, b_bf):
    return jax.lax.dot(a.astype(BF), b_bf, preferred_element_type=F32)


def _knn_body(xrow_ref, xt_ref, idx_ref, h6_ref, *, n):
    xr = xrow_ref[0]
    xt = xt_ref[0]
    blk = xr.shape[0]

    xr0 = xr[:, 0:1]
    xr1 = xr[:, 1:2]
    x0row = xt[0:1, :]
    x1row = xt[1:2, :]
    a0 = xr0.astype(BF).astype(F32)
    a1 = xr1.astype(BF).astype(F32)
    b0 = x0row.astype(BF).astype(F32)
    b1 = x1row.astype(BF).astype(F32)
    m = a0 * b0 + a1 * b1
    xxr = xr0 * xr0 + xr1 * xr1
    xxf = x0row * x0row + x1row * x1row
    work = (2.0 * m - xxr) - xxf

    iota = jax.lax.broadcasted_iota(jnp.int32, (blk, n), 1)
    idx_cols = []
    gathered = []
    for k in range(K_NN):
        mk = jnp.max(work, axis=1, keepdims=True)
        cand = jnp.where(work == mk, iota, n)
        jk = jnp.min(cand, axis=1, keepdims=True)
        onehot = iota == jk
        if k < 2:
            gathered.append((
                jnp.sum(jnp.where(onehot, x0row, 0.0), axis=1, keepdims=True),
                jnp.sum(jnp.where(onehot, x1row, 0.0), axis=1, keepdims=True)))
        idx_cols.append(jk)
        if k < K_NN - 1:
            work = jnp.where(onehot, -jnp.inf, work)
    idx_ref[0] = jnp.concatenate(idx_cols, axis=1)
    (g0x, g0y), (g1x, g1y) = gathered
    cov = jnp.concatenate(
        [g0x * g1x, g0x * g1y, g0y * g1x, g0y * g1y], axis=1)
    h6_ref[0] = jnp.concatenate([xr, cov], axis=1)


def _knn(x):
    b, n, _ = x.shape
    xt = jnp.swapaxes(x, 1, 2)
    blk = 256
    return pl.pallas_call(
        functools.partial(_knn_body, n=n),
        grid=(b, n // blk),
        in_specs=[
            pl.BlockSpec((1, blk, 2), lambda i, j: (i, j, 0)),
            pl.BlockSpec((1, 2, n), lambda i, j: (i, 0, 0)),
        ],
        out_specs=[
            pl.BlockSpec((1, blk, K_NN), lambda i, j: (i, j, 0)),
            pl.BlockSpec((1, blk, 6), lambda i, j: (i, j, 0)),
        ],
        out_shape=[
            jax.ShapeDtypeStruct((b, n, K_NN), jnp.int32),
            jax.ShapeDtypeStruct((b, n, 6), F32),
        ],
        interpret=_INTERPRET,
    )(x, xt)


def _mlp3_body(x_ref, w1_ref, b1_ref, w2_ref, b2_ref, w3_ref, b3_ref, o_ref):
    a = _relu(_bdot(x_ref[...], w1_ref[...]) + b1_ref[...])
    a = _relu(_bdot(a, w2_ref[...]) + b2_ref[...])
    a = _relu(_bdot(a, w3_ref[...]) + b3_ref[...])
    o_ref[...] = a


def _mlp3(x, w1t, b1, w2t, b2, w3t, b3):
    rows, cin = x.shape
    cout = w3t.shape[1]
    blk = 1024
    full = lambda a: pl.BlockSpec(a.shape, lambda i: (0,) * a.ndim)
    return pl.pallas_call(
        _mlp3_body,
        grid=(rows // blk,),
        in_specs=[
            pl.BlockSpec((blk, cin), lambda i: (i, 0)),
            full(w1t), full(b1[None]),
            full(w2t), full(b2[None]),
            full(w3t), full(b3[None]),
        ],
        out_specs=pl.BlockSpec((blk, cout), lambda i: (i, 0)),
        out_shape=jax.ShapeDtypeStruct((rows, cout), F32),
        interpret=_INTERPRET,
    )(x, w1t, b1[None], w2t, b2[None], w3t, b3[None])


def _lmax_body(idx_ref, h_ref, o_ref, *, n):
    idxb = idx_ref[0]
    hf = h_ref[0]
    blk = idxb.shape[0]
    iota = jax.lax.broadcasted_iota(jnp.int32, (blk, n), 1)
    acc = None
    for k in range(K_NN):
        oh = (iota == idxb[:, k:k + 1]).astype(F32)
        gk = _dot(oh, hf)
        acc = gk if acc is None else jnp.maximum(acc, gk)
    o_ref[0] = acc


def _lmax(idx, h3):
    b, n, c = h3.shape
    blk = 256
    return pl.pallas_call(
        functools.partial(_lmax_body, n=n),
        grid=(b, n // blk),
        in_specs=[
            pl.BlockSpec((1, blk, K_NN), lambda i, j: (i, j, 0)),
            pl.BlockSpec((1, n, c), lambda i, j: (i, 0, 0)),
        ],
        out_specs=pl.BlockSpec((1, blk, c), lambda i, j: (i, j, 0)),
        out_shape=jax.ShapeDtypeStruct((b, n, c), F32),
        interpret=_INTERPRET,
    )(idx, h3).reshape(b * n, c)


def _lin2_body(x_ref, l_ref, lb_ref, c_ref, cb_ref, o_ref):
    ta = _bdot(x_ref[...], l_ref[...]) + lb_ref[...]
    o_ref[...] = _relu(_bdot(ta, c_ref[...]) + cb_ref[...])


def _lin2(x, l1t, lb1, c1t, cb1):
    rows = x.shape[0]
    cout = c1t.shape[1]
    blk = 1024
    full = lambda a: pl.BlockSpec(a.shape, lambda i: (0,) * a.ndim)
    return pl.pallas_call(
        _lin2_body,
        grid=(rows // blk,),
        in_specs=[
            pl.BlockSpec((blk, x.shape[1]), lambda i: (i, 0)),
            full(l1t), full(lb1[None]), full(c1t), full(cb1[None]),
        ],
        out_specs=pl.BlockSpec((blk, cout), lambda i: (i, 0)),
        out_shape=jax.ShapeDtypeStruct((rows, cout), F32),
        interpret=_INTERPRET,
    )(x, l1t, lb1[None], c1t, cb1[None])


def _head_body(q_ref, l2t_ref, lb2_ref, c2t_ref, cb2_ref, m1t_ref, mb1_ref,
               m2t_ref, mb2_ref, f11at_ref, fb11_ref, f21at_ref, fb21_ref,
               feat_ref, v1_ref, u1_ref):
    z = q_ref[0]
    tb = _bdot(z, l2t_ref[...]) + lb2_ref[...]
    y = _bdot(tb, c2t_ref[...]) + cb2_ref[...]
    mx = jnp.max(y, axis=0, keepdims=True)
    t = _relu(_bdot(mx, m1t_ref[...]) + mb1_ref[...])
    feat = _bdot(t, m2t_ref[...]) + mb2_ref[...]
    feat_ref[0] = feat
    v1_ref[0] = _bdot(feat, f11at_ref[...]) + fb11_ref[...]
    u1_ref[0] = _bdot(feat, f21at_ref[...]) + fb21_ref[...]


def _head(q3, l2t, lb2, c2t, cb2, m1t, mb1, m2t, mb2, f11at, fb11,
          f21at, fb21):
    b, n, c = q3.shape
    full = lambda a: pl.BlockSpec(a.shape, lambda i: (0,) * a.ndim)
    return pl.pallas_call(
        _head_body,
        grid=(b,),
        in_specs=[
            pl.BlockSpec((1, n, c), lambda i: (i, 0, 0)),
            full(l2t), full(lb2[None]), full(c2t), full(cb2[None]),
            full(m1t), full(mb1[None]), full(m2t), full(mb2[None]),
            full(f11at), full(fb11[None]), full(f21at), full(fb21[None]),
        ],
        out_specs=[
            pl.BlockSpec((1, 1, 512), lambda i: (i, 0, 0)),
            pl.BlockSpec((1, 1, 512), lambda i: (i, 0, 0)),
            pl.BlockSpec((1, 1, 512), lambda i: (i, 0, 0)),
        ],
        out_shape=[
            jax.ShapeDtypeStruct((b, 1, 512), F32),
            jax.ShapeDtypeStruct((b, 1, 512), F32),
            jax.ShapeDtypeStruct((b, 1, 512), F32),
        ],
        interpret=_INTERPRET,
    )(q3, l2t, lb2[None], c2t, cb2[None], m1t, mb1[None], m2t, mb2[None],
      f11at, fb11[None], f21at, fb21[None])


def _fold_body(v1_ref, u1_ref, wg_ref, f12t_ref, fb12_ref, f13t_ref, fb13_ref,
               f21bt_ref, f22t_ref, fb22_ref, f23t_ref, fb23_ref, o_ref,
               *, blk):
    j = pl.program_id(1)
    step = jnp.float32(1.0) / jnp.float32(M_PTS - 1)
    gcol = (jax.lax.broadcasted_iota(jnp.int32, (blk, 1), 0)
            + j * blk).astype(F32) * step
    gb = gcol.astype(BF).astype(F32)
    f1a = _relu(v1_ref[0] + wg_ref[...] * gb)
    h1 = _relu(_bdot(f1a, f12t_ref[...]) + fb12_ref[...])
    f1 = _bdot(h1, f13t_ref[...]) + fb13_ref[...]
    g1a = _relu(u1_ref[0] + _bdot(f1, f21bt_ref[...]))
    h2 = _relu(_bdot(g1a, f22t_ref[...]) + fb22_ref[...])
    f2 = _bdot(h2, f23t_ref[...]) + fb23_ref[...]
    o_ref[0] = f2


def _fold(v1, u1, wg, f12t, fb12, f13t, fb13, f21bt, f22t, fb22, f23t, fb23):
    b = v1.shape[0]
    blk = 512
    full = lambda a: pl.BlockSpec(a.shape, lambda i, j: (0,) * a.ndim)
    row = lambda a: pl.BlockSpec((1, 1, a.shape[2]), lambda i, j: (i, 0, 0))
    return pl.pallas_call(
        functools.partial(_fold_body, blk=blk),
        grid=(b, M_PTS // blk),
        in_specs=[
            row(v1), row(u1), full(wg[None]),
            full(f12t), full(fb12[None]), full(f13t), full(fb13[None]),
            full(f21bt), full(f22t), full(fb22[None]),
            full(f23t), full(fb23[None]),
        ],
        out_specs=pl.BlockSpec((1, blk, 2), lambda i, j: (i, j, 0)),
        out_shape=jax.ShapeDtypeStruct((b, M_PTS, 2), F32),
        interpret=_INTERPRET,
    )(v1, u1, wg[None], f12t, fb12[None], f13t, fb13[None], f21bt,
      f22t, fb22[None], f23t, fb23[None])


def kernel(input, W1, b1, W2, b2, W3, b3, L1, lb1, C1, cb1, L2, lb2, C2, cb2,
           M1, mb1, M2, mb2, F11, fb11, F12, fb12, F13, fb13, F21, fb21,
           F22, fb22, F23, fb23):
    b, n, _ = input.shape
    tb = lambda w: w.T.astype(BF)

    wg = F11[:, 512].astype(BF).astype(F32)

    idx, h6 = _knn(input)
    h = _mlp3(h6.reshape(b * n, 6), tb(W1), b1, tb(W2), b2, tb(W3), b3)
    t1 = _lmax(idx, h.reshape(b, n, 64))
    q = _lin2(t1, tb(L1), lb1, tb(C1), cb1)
    t2 = _lmax(idx, q.reshape(b, n, 128))
    feat, v1, u1 = _head(t2.reshape(b, n, 128), tb(L2), lb2, tb(C2), cb2,
                         tb(M1), mb1, tb(M2), mb2,
                         tb(F11[:, :512]), fb11, tb(F21[:, :512]), fb21)
    out = _fold(v1, u1, wg, tb(F12), fb12, tb(F13), fb13,
                tb(F21[:, 512:514]), tb(F22), fb22, tb(F23), fb23)
    return out, feat


# speedup vs baseline: 20.2368x; 1.3121x over previous
import functools

import jax
import jax.numpy as jnp
from jax import lax
from jax.experimental import pallas as pl
from jax.experimental.pallas import tpu as pltpu
from jax.experimental.pallas import tpu_sc as plsc

_INTERPRET = False

K_NN = 16
M_PTS = 2048
F32 = jnp.float32
BF = jnp.bfloat16


def _relu(x):
    return jnp.maximum(x, 0.0)


def _dot(a, b):
    return jax.lax.dot(a, b, preferred_element_type=F32)


def _bdot(a, b_bf):
    return jax.lax.dot(a.astype(BF), b_bf, preferred_element_type=F32)


def _knn_body(xrow_ref, xt_ref, idx_ref, h6_ref, *, n):
    bofs = pl.program_id(0) * n
    xr = xrow_ref[0]
    xt = xt_ref[0]
    blk = xr.shape[0]

    xr0 = xr[:, 0:1]
    xr1 = xr[:, 1:2]
    x0row = xt[0:1, :]
    x1row = xt[1:2, :]
    a0 = xr0.astype(BF).astype(F32)
    a1 = xr1.astype(BF).astype(F32)
    b0 = x0row.astype(BF).astype(F32)
    b1 = x1row.astype(BF).astype(F32)
    m = a0 * b0 + a1 * b1
    xxr = xr0 * xr0 + xr1 * xr1
    xxf = x0row * x0row + x1row * x1row
    work = (2.0 * m - xxr) - xxf

    iota = jax.lax.broadcasted_iota(jnp.int32, (blk, n), 1)
    idx_cols = []
    gathered = []
    for k in range(K_NN):
        mk = jnp.max(work, axis=1, keepdims=True)
        cand = jnp.where(work == mk, iota, n)
        jk = jnp.min(cand, axis=1, keepdims=True)
        onehot = iota == jk
        if k < 2:
            gathered.append((
                jnp.sum(jnp.where(onehot, x0row, 0.0), axis=1, keepdims=True),
                jnp.sum(jnp.where(onehot, x1row, 0.0), axis=1, keepdims=True)))
        idx_cols.append(jk + bofs)
        if k < K_NN - 1:
            work = jnp.where(onehot, -jnp.inf, work)
    idx_ref[0] = jnp.concatenate(idx_cols, axis=1)
    (g0x, g0y), (g1x, g1y) = gathered
    cov = jnp.concatenate(
        [g0x * g1x, g0x * g1y, g0y * g1x, g0y * g1y], axis=1)
    h6_ref[0] = jnp.concatenate([xr, cov], axis=1)


def _knn(x):
    b, n, _ = x.shape
    xt = jnp.swapaxes(x, 1, 2)
    blk = 256
    return pl.pallas_call(
        functools.partial(_knn_body, n=n),
        grid=(b, n // blk),
        in_specs=[
            pl.BlockSpec((1, blk, 2), lambda i, j: (i, j, 0)),
            pl.BlockSpec((1, 2, n), lambda i, j: (i, 0, 0)),
        ],
        out_specs=[
            pl.BlockSpec((1, blk, K_NN), lambda i, j: (i, j, 0)),
            pl.BlockSpec((1, blk, 6), lambda i, j: (i, j, 0)),
        ],
        out_shape=[
            jax.ShapeDtypeStruct((b, n, K_NN), jnp.int32),
            jax.ShapeDtypeStruct((b, n, 6), F32),
        ],
        interpret=_INTERPRET,
    )(x, xt)


def _mlp3_body(x_ref, w1_ref, b1_ref, w2_ref, b2_ref, w3_ref, b3_ref, o_ref):
    a = _relu(_bdot(x_ref[...], w1_ref[...]) + b1_ref[...])
    a = _relu(_bdot(a, w2_ref[...]) + b2_ref[...])
    a = _relu(_bdot(a, w3_ref[...]) + b3_ref[...])
    o_ref[...] = jnp.concatenate([a, jnp.zeros_like(a)], axis=1)


def _mlp3(x, w1t, b1, w2t, b2, w3t, b3):
    rows, cin = x.shape
    cout = 2 * w3t.shape[1]
    blk = 1024
    full = lambda a: pl.BlockSpec(a.shape, lambda i: (0,) * a.ndim)
    return pl.pallas_call(
        _mlp3_body,
        grid=(rows // blk,),
        in_specs=[
            pl.BlockSpec((blk, cin), lambda i: (i, 0)),
            full(w1t), full(b1[None]),
            full(w2t), full(b2[None]),
            full(w3t), full(b3[None]),
        ],
        out_specs=pl.BlockSpec((blk, cout), lambda i: (i, 0)),
        out_shape=jax.ShapeDtypeStruct((rows, cout), F32),
        interpret=_INTERPRET,
    )(x, w1t, b1[None], w2t, b2[None], w3t, b3[None])


def _lmax_body(idx_ref, h_ref, o_ref, *, n):
    idxb = idx_ref[0]
    hf = h_ref[0]
    blk = idxb.shape[0]
    iota = jax.lax.broadcasted_iota(jnp.int32, (blk, n), 1)
    acc = None
    for k in range(K_NN):
        oh = (iota == idxb[:, k:k + 1]).astype(F32)
        gk = _dot(oh, hf)
        acc = gk if acc is None else jnp.maximum(acc, gk)
    o_ref[0] = acc


def _lmax(idx, h3):
    b, n, c = h3.shape
    blk = 256
    return pl.pallas_call(
        functools.partial(_lmax_body, n=n),
        grid=(b, n // blk),
        in_specs=[
            pl.BlockSpec((1, blk, K_NN), lambda i, j: (i, j, 0)),
            pl.BlockSpec((1, n, c), lambda i, j: (i, 0, 0)),
        ],
        out_specs=pl.BlockSpec((1, blk, c), lambda i, j: (i, j, 0)),
        out_shape=jax.ShapeDtypeStruct((b, n, c), F32),
        interpret=_INTERPRET,
    )(idx, h3).reshape(b * n, c)


def _lmax_sc(idx_flat, h, c):
    rows, cw = h.shape
    info = plsc.get_sparse_core_info()
    nw = info.num_cores * info.num_subcores
    ppw = rows // nw
    p = 4096 // cw
    ngs = (p * K_NN) // 128
    nchunks = ppw // p
    mesh = plsc.VectorSubcoreMesh(core_axis_name="c", subcore_axis_name="s")

    @functools.partial(
        pl.kernel, mesh=mesh,
        out_type=jax.ShapeDtypeStruct((rows, c), F32),
        scratch_types=[
            pltpu.VMEM((ngs, 128), jnp.int32),
            pltpu.VMEM((p * K_NN, cw), F32),
            pltpu.VMEM((p, c), F32),
            pltpu.SemaphoreType.DMA,
        ])
    def k(idx_hbm, h_hbm, out_hbm, idxc_v, rows_v, out_v, sem):
        wid = lax.axis_index("s") * info.num_cores + lax.axis_index("c")
        base_pt = wid * ppw

        def chunk_body(ci, carry):
            pt0 = base_pt + ci * p
            for s in range(ngs):
                pltpu.sync_copy(
                    idx_hbm.at[pl.ds(pt0 * K_NN + s * 128, 128)],
                    idxc_v.at[s])
            copies = [
                pltpu.async_copy(h_hbm.at[idxc_v.at[s]],
                                 rows_v.at[pl.ds(s * 128, 128)], sem)
                for s in range(ngs)]
            for cp in copies:
                cp.wait()

            def pt_body(pi, carry2):
                for cs in range(c // 16):
                    sl = pl.ds(cs * 16, 16)
                    acc = rows_v[pi * K_NN, sl]
                    for kk in range(1, K_NN):
                        acc = jnp.maximum(acc, rows_v[pi * K_NN + kk, sl])
                    out_v[pi, sl] = acc
                return carry2

            lax.fori_loop(0, p, pt_body, 0)
            pltpu.sync_copy(out_v, out_hbm.at[pl.ds(pt0, p)])
            return carry

        lax.fori_loop(0, nchunks, chunk_body, 0)

    return k(idx_flat, h)


def _lin2_body(x_ref, l_ref, lb_ref, c_ref, cb_ref, o_ref):
    ta = _bdot(x_ref[...], l_ref[...]) + lb_ref[...]
    o_ref[...] = _relu(_bdot(ta, c_ref[...]) + cb_ref[...])


def _lin2(x, l1t, lb1, c1t, cb1):
    rows = x.shape[0]
    cout = c1t.shape[1]
    blk = 1024
    full = lambda a: pl.BlockSpec(a.shape, lambda i: (0,) * a.ndim)
    return pl.pallas_call(
        _lin2_body,
        grid=(rows // blk,),
        in_specs=[
            pl.BlockSpec((blk, x.shape[1]), lambda i: (i, 0)),
            full(l1t), full(lb1[None]), full(c1t), full(cb1[None]),
        ],
        out_specs=pl.BlockSpec((blk, cout), lambda i: (i, 0)),
        out_shape=jax.ShapeDtypeStruct((rows, cout), F32),
        interpret=_INTERPRET,
    )(x, l1t, lb1[None], c1t, cb1[None])


def _head_body(q_ref, l2t_ref, lb2_ref, c2t_ref, cb2_ref, m1t_ref, mb1_ref,
               m2t_ref, mb2_ref, f11at_ref, fb11_ref, f21at_ref, fb21_ref,
               feat_ref, v1_ref, u1_ref):
    z = q_ref[0]
    tb = _bdot(z, l2t_ref[...]) + lb2_ref[...]
    y = _bdot(tb, c2t_ref[...]) + cb2_ref[...]
    mx = jnp.max(y, axis=0, keepdims=True)
    t = _relu(_bdot(mx, m1t_ref[...]) + mb1_ref[...])
    feat = _bdot(t, m2t_ref[...]) + mb2_ref[...]
    feat_ref[0] = feat
    v1_ref[0] = _bdot(feat, f11at_ref[...]) + fb11_ref[...]
    u1_ref[0] = _bdot(feat, f21at_ref[...]) + fb21_ref[...]


def _head(q3, l2t, lb2, c2t, cb2, m1t, mb1, m2t, mb2, f11at, fb11,
          f21at, fb21):
    b, n, c = q3.shape
    full = lambda a: pl.BlockSpec(a.shape, lambda i: (0,) * a.ndim)
    return pl.pallas_call(
        _head_body,
        grid=(b,),
        in_specs=[
            pl.BlockSpec((1, n, c), lambda i: (i, 0, 0)),
            full(l2t), full(lb2[None]), full(c2t), full(cb2[None]),
            full(m1t), full(mb1[None]), full(m2t), full(mb2[None]),
            full(f11at), full(fb11[None]), full(f21at), full(fb21[None]),
        ],
        out_specs=[
            pl.BlockSpec((1, 1, 512), lambda i: (i, 0, 0)),
            pl.BlockSpec((1, 1, 512), lambda i: (i, 0, 0)),
            pl.BlockSpec((1, 1, 512), lambda i: (i, 0, 0)),
        ],
        out_shape=[
            jax.ShapeDtypeStruct((b, 1, 512), F32),
            jax.ShapeDtypeStruct((b, 1, 512), F32),
            jax.ShapeDtypeStruct((b, 1, 512), F32),
        ],
        interpret=_INTERPRET,
    )(q3, l2t, lb2[None], c2t, cb2[None], m1t, mb1[None], m2t, mb2[None],
      f11at, fb11[None], f21at, fb21[None])


def _fold_body(v1_ref, u1_ref, wg_ref, f12t_ref, fb12_ref, f13t_ref, fb13_ref,
               f21bt_ref, f22t_ref, fb22_ref, f23t_ref, fb23_ref, o_ref,
               *, blk):
    j = pl.program_id(1)
    step = jnp.float32(1.0) / jnp.float32(M_PTS - 1)
    gcol = (jax.lax.broadcasted_iota(jnp.int32, (blk, 1), 0)
            + j * blk).astype(F32) * step
    gb = gcol.astype(BF).astype(F32)
    f1a = _relu(v1_ref[0] + wg_ref[...] * gb)
    h1 = _relu(_bdot(f1a, f12t_ref[...]) + fb12_ref[...])
    f1 = _bdot(h1, f13t_ref[...]) + fb13_ref[...]
    g1a = _relu(u1_ref[0] + _bdot(f1, f21bt_ref[...]))
    h2 = _relu(_bdot(g1a, f22t_ref[...]) + fb22_ref[...])
    f2 = _bdot(h2, f23t_ref[...]) + fb23_ref[...]
    o_ref[0] = f2


def _fold(v1, u1, wg, f12t, fb12, f13t, fb13, f21bt, f22t, fb22, f23t, fb23):
    b = v1.shape[0]
    blk = 512
    full = lambda a: pl.BlockSpec(a.shape, lambda i, j: (0,) * a.ndim)
    row = lambda a: pl.BlockSpec((1, 1, a.shape[2]), lambda i, j: (i, 0, 0))
    return pl.pallas_call(
        functools.partial(_fold_body, blk=blk),
        grid=(b, M_PTS // blk),
        in_specs=[
            row(v1), row(u1), full(wg[None]),
            full(f12t), full(fb12[None]), full(f13t), full(fb13[None]),
            full(f21bt), full(f22t), full(fb22[None]),
            full(f23t), full(fb23[None]),
        ],
        out_specs=pl.BlockSpec((1, blk, 2), lambda i, j: (i, j, 0)),
        out_shape=jax.ShapeDtypeStruct((b, M_PTS, 2), F32),
        interpret=_INTERPRET,
    )(v1, u1, wg[None], f12t, fb12[None], f13t, fb13[None], f21bt,
      f22t, fb22[None], f23t, fb23[None])


def kernel(input, W1, b1, W2, b2, W3, b3, L1, lb1, C1, cb1, L2, lb2, C2, cb2,
           M1, mb1, M2, mb2, F11, fb11, F12, fb12, F13, fb13, F21, fb21,
           F22, fb22, F23, fb23):
    b, n, _ = input.shape
    tb = lambda w: w.T.astype(BF)

    wg = F11[:, 512].astype(BF).astype(F32)

    idx, h6 = _knn(input)
    idx_flat = idx.reshape(b * n * K_NN)
    h = _mlp3(h6.reshape(b * n, 6), tb(W1), b1, tb(W2), b2, tb(W3), b3)
    t1 = _lmax_sc(idx_flat, h, 64)
    q = _lin2(t1, tb(L1), lb1, tb(C1), cb1)
    t2 = _lmax_sc(idx_flat, q, 128)
    feat, v1, u1 = _head(t2.reshape(b, n, 128), tb(L2), lb2, tb(C2), cb2,
                         tb(M1), mb1, tb(M2), mb2,
                         tb(F11[:, :512]), fb11, tb(F21[:, :512]), fb21)
    out = _fold(v1, u1, wg, tb(F12), fb12, tb(F13), fb13,
                tb(F21[:, 512:514]), tb(F22), fb22, tb(F23), fb23)
    return out, feat


# speedup vs baseline: 20.5405x; 1.0150x over previous
import functools

import jax
import jax.numpy as jnp
from jax import lax
from jax.experimental import pallas as pl
from jax.experimental.pallas import tpu as pltpu
from jax.experimental.pallas import tpu_sc as plsc

_INTERPRET = False

K_NN = 16
M_PTS = 2048
F32 = jnp.float32
BF = jnp.bfloat16


def _relu(x):
    return jnp.maximum(x, 0.0)


def _dot(a, b):
    return jax.lax.dot(a, b, preferred_element_type=F32)


def _bdot(a, b_bf):
    return jax.lax.dot(a.astype(BF), b_bf, preferred_element_type=F32)


def _knn_body(xrow_ref, xt_ref, w1_ref, b1_ref, w2_ref, b2_ref, w3_ref,
              b3_ref, idx_ref, h_ref, *, n):
    bofs = pl.program_id(0) * n
    xr = xrow_ref[0]
    xt = xt_ref[0]
    blk = xr.shape[0]

    xr0 = xr[:, 0:1]
    xr1 = xr[:, 1:2]
    x0row = xt[0:1, :]
    x1row = xt[1:2, :]
    m = jax.lax.dot(xr.astype(BF), xt.astype(BF),
                    preferred_element_type=F32)
    xxr = xr0 * xr0 + xr1 * xr1
    xxf = x0row * x0row + x1row * x1row
    work = (2.0 * m - xxr) - xxf

    iota = jax.lax.broadcasted_iota(jnp.int32, (blk, n), 1)
    idx_cols = []
    gathered = []
    for k in range(K_NN):
        mk = jnp.max(work, axis=1, keepdims=True)
        cand = jnp.where(work == mk, iota, n)
        jk = jnp.min(cand, axis=1, keepdims=True)
        onehot = iota == jk
        if k < 2:
            gathered.append((
                jnp.sum(jnp.where(onehot, x0row, 0.0), axis=1, keepdims=True),
                jnp.sum(jnp.where(onehot, x1row, 0.0), axis=1, keepdims=True)))
        idx_cols.append(jk + bofs)
        if k < K_NN - 1:
            work = jnp.where(onehot, -jnp.inf, work)
    idx_ref[0] = jnp.concatenate(idx_cols, axis=1)
    (g0x, g0y), (g1x, g1y) = gathered
    cov = jnp.concatenate(
        [g0x * g1x, g0x * g1y, g0y * g1x, g0y * g1y], axis=1)
    h6 = jnp.concatenate([xr, cov], axis=1)
    a = _relu(_bdot(h6, w1_ref[...]) + b1_ref[...])
    a = _relu(_bdot(a, w2_ref[...]) + b2_ref[...])
    a = _relu(_bdot(a, w3_ref[...]) + b3_ref[...])
    h_ref[0] = jnp.concatenate([a, jnp.zeros_like(a)], axis=1)


def _knn(x, w1t, b1, w2t, b2, w3t, b3):
    b, n, _ = x.shape
    xt = jnp.swapaxes(x, 1, 2)
    blk = 256
    full = lambda a: pl.BlockSpec(a.shape, lambda i, j: (0,) * a.ndim)
    return pl.pallas_call(
        functools.partial(_knn_body, n=n),
        grid=(b, n // blk),
        in_specs=[
            pl.BlockSpec((1, blk, 2), lambda i, j: (i, j, 0)),
            pl.BlockSpec((1, 2, n), lambda i, j: (i, 0, 0)),
            full(w1t), full(b1[None]), full(w2t), full(b2[None]),
            full(w3t), full(b3[None]),
        ],
        out_specs=[
            pl.BlockSpec((1, blk, K_NN), lambda i, j: (i, j, 0)),
            pl.BlockSpec((1, blk, 128), lambda i, j: (i, j, 0)),
        ],
        out_shape=[
            jax.ShapeDtypeStruct((b, n, K_NN), jnp.int32),
            jax.ShapeDtypeStruct((b, n, 128), F32),
        ],
        interpret=_INTERPRET,
    )(x, xt, w1t, b1[None], w2t, b2[None], w3t, b3[None])


def _mlp3_body(x_ref, w1_ref, b1_ref, w2_ref, b2_ref, w3_ref, b3_ref, o_ref):
    a = _relu(_bdot(x_ref[...], w1_ref[...]) + b1_ref[...])
    a = _relu(_bdot(a, w2_ref[...]) + b2_ref[...])
    a = _relu(_bdot(a, w3_ref[...]) + b3_ref[...])
    o_ref[...] = jnp.concatenate([a, jnp.zeros_like(a)], axis=1)


def _mlp3(x, w1t, b1, w2t, b2, w3t, b3):
    rows, cin = x.shape
    cout = 2 * w3t.shape[1]
    blk = 1024
    full = lambda a: pl.BlockSpec(a.shape, lambda i: (0,) * a.ndim)
    return pl.pallas_call(
        _mlp3_body,
        grid=(rows // blk,),
        in_specs=[
            pl.BlockSpec((blk, cin), lambda i: (i, 0)),
            full(w1t), full(b1[None]),
            full(w2t), full(b2[None]),
            full(w3t), full(b3[None]),
        ],
        out_specs=pl.BlockSpec((blk, cout), lambda i: (i, 0)),
        out_shape=jax.ShapeDtypeStruct((rows, cout), F32),
        interpret=_INTERPRET,
    )(x, w1t, b1[None], w2t, b2[None], w3t, b3[None])


def _lmax_body(idx_ref, h_ref, o_ref, *, n):
    idxb = idx_ref[0]
    hf = h_ref[0]
    blk = idxb.shape[0]
    iota = jax.lax.broadcasted_iota(jnp.int32, (blk, n), 1)
    acc = None
    for k in range(K_NN):
        oh = (iota == idxb[:, k:k + 1]).astype(F32)
        gk = _dot(oh, hf)
        acc = gk if acc is None else jnp.maximum(acc, gk)
    o_ref[0] = acc


def _lmax(idx, h3):
    b, n, c = h3.shape
    blk = 256
    return pl.pallas_call(
        functools.partial(_lmax_body, n=n),
        grid=(b, n // blk),
        in_specs=[
            pl.BlockSpec((1, blk, K_NN), lambda i, j: (i, j, 0)),
            pl.BlockSpec((1, n, c), lambda i, j: (i, 0, 0)),
        ],
        out_specs=pl.BlockSpec((1, blk, c), lambda i, j: (i, j, 0)),
        out_shape=jax.ShapeDtypeStruct((b, n, c), F32),
        interpret=_INTERPRET,
    )(idx, h3).reshape(b * n, c)


def _lmax_sc(idx_flat, h, c):
    rows, cw = h.shape
    info = plsc.get_sparse_core_info()
    nw = info.num_cores * info.num_subcores
    ppw = rows // nw
    p = 4096 // cw
    ngs = (p * K_NN) // 128
    nchunks = ppw // p
    mesh = plsc.VectorSubcoreMesh(core_axis_name="c", subcore_axis_name="s")

    @functools.partial(
        pl.kernel, mesh=mesh,
        out_type=jax.ShapeDtypeStruct((rows, c), F32),
        scratch_types=[
            pltpu.VMEM((ngs, 128), jnp.int32),
            pltpu.VMEM((p * K_NN, cw), F32),
            pltpu.VMEM((p, c), F32),
            pltpu.SemaphoreType.DMA,
        ])
    def k(idx_hbm, h_hbm, out_hbm, idxc_v, rows_v, out_v, sem):
        wid = lax.axis_index("s") * info.num_cores + lax.axis_index("c")
        base_pt = wid * ppw

        def chunk_body(ci, carry):
            pt0 = base_pt + ci * p
            for s in range(ngs):
                pltpu.sync_copy(
                    idx_hbm.at[pl.ds(pt0 * K_NN + s * 128, 128)],
                    idxc_v.at[s])
            copies = [
                pltpu.async_copy(h_hbm.at[idxc_v.at[s]],
                                 rows_v.at[pl.ds(s * 128, 128)], sem)
                for s in range(ngs)]
            for cp in copies:
                cp.wait()

            def pt_body(pi, carry2):
                for cs in range(c // 16):
                    sl = pl.ds(cs * 16, 16)
                    acc = rows_v[pi * K_NN, sl]
                    for kk in range(1, K_NN):
                        acc = jnp.maximum(acc, rows_v[pi * K_NN + kk, sl])
                    out_v[pi, sl] = acc
                return carry2

            lax.fori_loop(0, p, pt_body, 0)
            pltpu.sync_copy(out_v, out_hbm.at[pl.ds(pt0, p)])
            return carry

        lax.fori_loop(0, nchunks, chunk_body, 0)

    return k(idx_flat, h)


def _lin2_body(x_ref, l_ref, lb_ref, c_ref, cb_ref, o_ref):
    ta = _bdot(x_ref[...], l_ref[...]) + lb_ref[...]
    o_ref[...] = _relu(_bdot(ta, c_ref[...]) + cb_ref[...])


def _lin2(x, l1t, lb1, c1t, cb1):
    rows = x.shape[0]
    cout = c1t.shape[1]
    blk = 1024
    full = lambda a: pl.BlockSpec(a.shape, lambda i: (0,) * a.ndim)
    return pl.pallas_call(
        _lin2_body,
        grid=(rows // blk,),
        in_specs=[
            pl.BlockSpec((blk, x.shape[1]), lambda i: (i, 0)),
            full(l1t), full(lb1[None]), full(c1t), full(cb1[None]),
        ],
        out_specs=pl.BlockSpec((blk, cout), lambda i: (i, 0)),
        out_shape=jax.ShapeDtypeStruct((rows, cout), F32),
        interpret=_INTERPRET,
    )(x, l1t, lb1[None], c1t, cb1[None])


def _head_body(q_ref, l2t_ref, lb2_ref, c2t_ref, cb2_ref, m1t_ref, mb1_ref,
               m2t_ref, mb2_ref, f11at_ref, fb11_ref, f21at_ref, fb21_ref,
               feat_ref, v1_ref, u1_ref):
    z = q_ref[0]
    tb = _bdot(z, l2t_ref[...]) + lb2_ref[...]
    y = _bdot(tb, c2t_ref[...]) + cb2_ref[...]
    mx = jnp.max(y, axis=0, keepdims=True)
    t = _relu(_bdot(mx, m1t_ref[...]) + mb1_ref[...])
    feat = _bdot(t, m2t_ref[...]) + mb2_ref[...]
    feat_ref[0] = feat
    v1_ref[0] = _bdot(feat, f11at_ref[...]) + fb11_ref[...]
    u1_ref[0] = _bdot(feat, f21at_ref[...]) + fb21_ref[...]


def _head(q3, l2t, lb2, c2t, cb2, m1t, mb1, m2t, mb2, f11at, fb11,
          f21at, fb21):
    b, n, c = q3.shape
    full = lambda a: pl.BlockSpec(a.shape, lambda i: (0,) * a.ndim)
    return pl.pallas_call(
        _head_body,
        grid=(b,),
        in_specs=[
            pl.BlockSpec((1, n, c), lambda i: (i, 0, 0)),
            full(l2t), full(lb2[None]), full(c2t), full(cb2[None]),
            full(m1t), full(mb1[None]), full(m2t), full(mb2[None]),
            full(f11at), full(fb11[None]), full(f21at), full(fb21[None]),
        ],
        out_specs=[
            pl.BlockSpec((1, 1, 512), lambda i: (i, 0, 0)),
            pl.BlockSpec((1, 1, 512), lambda i: (i, 0, 0)),
            pl.BlockSpec((1, 1, 512), lambda i: (i, 0, 0)),
        ],
        out_shape=[
            jax.ShapeDtypeStruct((b, 1, 512), F32),
            jax.ShapeDtypeStruct((b, 1, 512), F32),
            jax.ShapeDtypeStruct((b, 1, 512), F32),
        ],
        interpret=_INTERPRET,
    )(q3, l2t, lb2[None], c2t, cb2[None], m1t, mb1[None], m2t, mb2[None],
      f11at, fb11[None], f21at, fb21[None])


def _fold_body(v1_ref, u1_ref, wg_ref, f12t_ref, fb12_ref, f13t_ref, fb13_ref,
               f21bt_ref, f22t_ref, fb22_ref, f23t_ref, fb23_ref, o_ref,
               *, blk):
    j = pl.program_id(1)
    step = jnp.float32(1.0) / jnp.float32(M_PTS - 1)
    gcol = (jax.lax.broadcasted_iota(jnp.int32, (blk, 1), 0)
            + j * blk).astype(F32) * step
    gb = gcol.astype(BF).astype(F32)
    f1a = _relu(v1_ref[0] + wg_ref[...] * gb)
    h1 = _relu(_bdot(f1a, f12t_ref[...]) + fb12_ref[...])
    f1 = _bdot(h1, f13t_ref[...]) + fb13_ref[...]
    g1a = _relu(u1_ref[0] + _bdot(f1, f21bt_ref[...]))
    h2 = _relu(_bdot(g1a, f22t_ref[...]) + fb22_ref[...])
    f2 = _bdot(h2, f23t_ref[...]) + fb23_ref[...]
    o_ref[0] = f2


def _fold(v1, u1, wg, f12t, fb12, f13t, fb13, f21bt, f22t, fb22, f23t, fb23):
    b = v1.shape[0]
    blk = 512
    full = lambda a: pl.BlockSpec(a.shape, lambda i, j: (0,) * a.ndim)
    row = lambda a: pl.BlockSpec((1, 1, a.shape[2]), lambda i, j: (i, 0, 0))
    return pl.pallas_call(
        functools.partial(_fold_body, blk=blk),
        grid=(b, M_PTS // blk),
        in_specs=[
            row(v1), row(u1), full(wg[None]),
            full(f12t), full(fb12[None]), full(f13t), full(fb13[None]),
            full(f21bt), full(f22t), full(fb22[None]),
            full(f23t), full(fb23[None]),
        ],
        out_specs=pl.BlockSpec((1, blk, 2), lambda i, j: (i, j, 0)),
        out_shape=jax.ShapeDtypeStruct((b, M_PTS, 2), F32),
        interpret=_INTERPRET,
    )(v1, u1, wg[None], f12t, fb12[None], f13t, fb13[None], f21bt,
      f22t, fb22[None], f23t, fb23[None])


def kernel(input, W1, b1, W2, b2, W3, b3, L1, lb1, C1, cb1, L2, lb2, C2, cb2,
           M1, mb1, M2, mb2, F11, fb11, F12, fb12, F13, fb13, F21, fb21,
           F22, fb22, F23, fb23):
    b, n, _ = input.shape
    tb = lambda w: w.T.astype(BF)

    wg = F11[:, 512].astype(BF).astype(F32)

    idx, h3 = _knn(input, tb(W1), b1, tb(W2), b2, tb(W3), b3)
    idx_flat = idx.reshape(b * n * K_NN)
    h = h3.reshape(b * n, 128)
    t1 = _lmax_sc(idx_flat, h, 64)
    q = _lin2(t1, tb(L1), lb1, tb(C1), cb1)
    t2 = _lmax_sc(idx_flat, q, 128)
    feat, v1, u1 = _head(t2.reshape(b, n, 128), tb(L2), lb2, tb(C2), cb2,
                         tb(M1), mb1, tb(M2), mb2,
                         tb(F11[:, :512]), fb11, tb(F21[:, :512]), fb21)
    out = _fold(v1, u1, wg, tb(F12), fb12, tb(F13), fb13,
                tb(F21[:, 512:514]), tb(F22), fb22, tb(F23), fb23)
    return out, feat


# speedup vs baseline: 23.1449x; 1.1268x over previous
import functools

import jax
import jax.numpy as jnp
from jax import lax
from jax.experimental import pallas as pl
from jax.experimental.pallas import tpu as pltpu
from jax.experimental.pallas import tpu_sc as plsc

_INTERPRET = False

K_NN = 16
M_PTS = 2048
F32 = jnp.float32
BF = jnp.bfloat16


def _relu(x):
    return jnp.maximum(x, 0.0)


def _dot(a, b):
    return jax.lax.dot(a, b, preferred_element_type=F32)


def _bdot(a, b_bf):
    return jax.lax.dot(a.astype(BF), b_bf, preferred_element_type=F32)


def _knn_body(xrow_ref, xt_ref, w1_ref, b1_ref, w2_ref, b2_ref, w3_ref,
              b3_ref, idx_ref, h_ref, *, n):
    bofs = pl.program_id(0) * n
    xr = xrow_ref[0]
    xt = xt_ref[0]
    blk = xr.shape[0]

    xr0 = xr[:, 0:1]
    xr1 = xr[:, 1:2]
    x0row = xt[0:1, :]
    x1row = xt[1:2, :]
    m = jax.lax.dot(xr.astype(BF), xt.astype(BF),
                    preferred_element_type=F32)
    xxr = xr0 * xr0 + xr1 * xr1
    xxf = x0row * x0row + x1row * x1row
    work = (2.0 * m - xxr) - xxf

    iota = jax.lax.broadcasted_iota(jnp.int32, (blk, n), 1)
    idx_cols = []
    gathered = []
    for k in range(K_NN):
        mk = jnp.max(work, axis=1, keepdims=True)
        cand = jnp.where(work == mk, iota, n)
        jk = jnp.min(cand, axis=1, keepdims=True)
        onehot = iota == jk
        if k < 2:
            gathered.append((
                jnp.sum(jnp.where(onehot, x0row, 0.0), axis=1, keepdims=True),
                jnp.sum(jnp.where(onehot, x1row, 0.0), axis=1, keepdims=True)))
        idx_cols.append(jk + bofs)
        if k < K_NN - 1:
            work = jnp.where(onehot, -jnp.inf, work)
    idx_ref[0] = jnp.concatenate(idx_cols, axis=1)
    (g0x, g0y), (g1x, g1y) = gathered
    cov = jnp.concatenate(
        [g0x * g1x, g0x * g1y, g0y * g1x, g0y * g1y], axis=1)
    h6 = jnp.concatenate([xr, cov], axis=1)
    a = _relu(_bdot(h6, w1_ref[...]) + b1_ref[...])
    a = _relu(_bdot(a, w2_ref[...]) + b2_ref[...])
    a = _relu(_bdot(a, w3_ref[...]) + b3_ref[...])
    h_ref[0] = jnp.concatenate([a, jnp.zeros_like(a)], axis=1)


def _knn(x, w1t, b1, w2t, b2, w3t, b3):
    b, n, _ = x.shape
    xt = jnp.swapaxes(x, 1, 2)
    blk = 256
    full = lambda a: pl.BlockSpec(a.shape, lambda i, j: (0,) * a.ndim)
    return pl.pallas_call(
        functools.partial(_knn_body, n=n),
        grid=(b, n // blk),
        in_specs=[
            pl.BlockSpec((1, blk, 2), lambda i, j: (i, j, 0)),
            pl.BlockSpec((1, 2, n), lambda i, j: (i, 0, 0)),
            full(w1t), full(b1[None]), full(w2t), full(b2[None]),
            full(w3t), full(b3[None]),
        ],
        out_specs=[
            pl.BlockSpec((1, blk, K_NN), lambda i, j: (i, j, 0)),
            pl.BlockSpec((1, blk, 128), lambda i, j: (i, j, 0)),
        ],
        out_shape=[
            jax.ShapeDtypeStruct((b, n, K_NN), jnp.int32),
            jax.ShapeDtypeStruct((b, n, 128), F32),
        ],
        interpret=_INTERPRET,
    )(x, xt, w1t, b1[None], w2t, b2[None], w3t, b3[None])


def _mlp3_body(x_ref, w1_ref, b1_ref, w2_ref, b2_ref, w3_ref, b3_ref, o_ref):
    a = _relu(_bdot(x_ref[...], w1_ref[...]) + b1_ref[...])
    a = _relu(_bdot(a, w2_ref[...]) + b2_ref[...])
    a = _relu(_bdot(a, w3_ref[...]) + b3_ref[...])
    o_ref[...] = jnp.concatenate([a, jnp.zeros_like(a)], axis=1)


def _mlp3(x, w1t, b1, w2t, b2, w3t, b3):
    rows, cin = x.shape
    cout = 2 * w3t.shape[1]
    blk = 1024
    full = lambda a: pl.BlockSpec(a.shape, lambda i: (0,) * a.ndim)
    return pl.pallas_call(
        _mlp3_body,
        grid=(rows // blk,),
        in_specs=[
            pl.BlockSpec((blk, cin), lambda i: (i, 0)),
            full(w1t), full(b1[None]),
            full(w2t), full(b2[None]),
            full(w3t), full(b3[None]),
        ],
        out_specs=pl.BlockSpec((blk, cout), lambda i: (i, 0)),
        out_shape=jax.ShapeDtypeStruct((rows, cout), F32),
        interpret=_INTERPRET,
    )(x, w1t, b1[None], w2t, b2[None], w3t, b3[None])


def _lmax_body(idx_ref, h_ref, o_ref, *, n):
    idxb = idx_ref[0]
    hf = h_ref[0]
    blk = idxb.shape[0]
    iota = jax.lax.broadcasted_iota(jnp.int32, (blk, n), 1)
    acc = None
    for k in range(K_NN):
        oh = (iota == idxb[:, k:k + 1]).astype(F32)
        gk = _dot(oh, hf)
        acc = gk if acc is None else jnp.maximum(acc, gk)
    o_ref[0] = acc


def _lmax(idx, h3):
    b, n, c = h3.shape
    blk = 256
    return pl.pallas_call(
        functools.partial(_lmax_body, n=n),
        grid=(b, n // blk),
        in_specs=[
            pl.BlockSpec((1, blk, K_NN), lambda i, j: (i, j, 0)),
            pl.BlockSpec((1, n, c), lambda i, j: (i, 0, 0)),
        ],
        out_specs=pl.BlockSpec((1, blk, c), lambda i, j: (i, j, 0)),
        out_shape=jax.ShapeDtypeStruct((b, n, c), F32),
        interpret=_INTERPRET,
    )(idx, h3).reshape(b * n, c)


def _lmax_sc(idx_flat, h, c):
    rows, cw = h.shape
    idx2d = idx_flat.reshape(-1, 128)
    info = plsc.get_sparse_core_info()
    nw = info.num_cores * info.num_subcores
    ppw = rows // nw
    p = 16
    ngs = (p * K_NN) // 128
    nchunks = ppw // p
    nb = 2
    mesh = plsc.VectorSubcoreMesh(core_axis_name="c", subcore_axis_name="s")

    @functools.partial(
        pl.kernel, mesh=mesh,
        out_type=jax.ShapeDtypeStruct((rows, c), F32),
        scratch_types=[
            pltpu.VMEM((nb, ngs, 128), jnp.int32),
            pltpu.VMEM((nb, p * K_NN, cw), F32),
            pltpu.VMEM((p, c), F32),
            pltpu.SemaphoreType.DMA,
            pltpu.SemaphoreType.DMA,
        ])
    def k(idx_hbm, h_hbm, out_hbm, idxc_v, rows_v, out_v, sem0, sem1):
        wid = lax.axis_index("s") * info.num_cores + lax.axis_index("c")
        base_pt = wid * ppw
        sems = [sem0, sem1]

        def fire(ci, bi):
            pltpu.sync_copy(idx_hbm.at[pl.ds(ci * ngs, ngs)], idxc_v.at[bi])
            for s in range(ngs):
                pltpu.async_copy(h_hbm.at[idxc_v.at[bi].at[s]],
                                 rows_v.at[bi].at[pl.ds(s * 128, 128)],
                                 sems[bi])

        def drain_compute(ci, bi):
            for s in range(ngs):
                pltpu.make_async_copy(
                    h_hbm.at[idxc_v.at[bi].at[s]],
                    rows_v.at[bi].at[pl.ds(s * 128, 128)],
                    sems[bi]).wait()

            def pt_body(pi, carry2):
                for cs in range(c // 16):
                    sl = pl.ds(cs * 16, 16)
                    acc = rows_v[bi, pi * K_NN, sl]
                    for kk in range(1, K_NN):
                        acc = jnp.maximum(acc, rows_v[bi, pi * K_NN + kk, sl])
                    out_v[pi, sl] = acc
                return carry2

            lax.fori_loop(0, p, pt_body, 0)
            pltpu.sync_copy(out_v, out_hbm.at[pl.ds(base_pt + ci * p, p)])

        c0 = base_pt // p
        for b_ in range(nb):
            fire(c0 + b_, b_)

        def ring_body(ci0, carry):
            for b_ in range(nb):
                ci = ci0 + b_
                drain_compute(ci, b_)
                next_ci = ci + nb

                @pl.when(next_ci < c0 + nchunks)
                def _():
                    fire(next_ci, b_)
            return carry

        lax.fori_loop(0, nchunks // nb, lambda t, cr:
                      ring_body(c0 + t * nb, cr), 0)

    return k(idx2d, h)


def _lin2_body(x_ref, l_ref, lb_ref, c_ref, cb_ref, o_ref):
    ta = _bdot(x_ref[...], l_ref[...]) + lb_ref[...]
    o_ref[...] = _relu(_bdot(ta, c_ref[...]) + cb_ref[...])


def _lin2(x, l1t, lb1, c1t, cb1):
    rows = x.shape[0]
    cout = c1t.shape[1]
    blk = 1024
    full = lambda a: pl.BlockSpec(a.shape, lambda i: (0,) * a.ndim)
    return pl.pallas_call(
        _lin2_body,
        grid=(rows // blk,),
        in_specs=[
            pl.BlockSpec((blk, x.shape[1]), lambda i: (i, 0)),
            full(l1t), full(lb1[None]), full(c1t), full(cb1[None]),
        ],
        out_specs=pl.BlockSpec((blk, cout), lambda i: (i, 0)),
        out_shape=jax.ShapeDtypeStruct((rows, cout), F32),
        interpret=_INTERPRET,
    )(x, l1t, lb1[None], c1t, cb1[None])


def _head_body(q_ref, l2t_ref, lb2_ref, c2t_ref, cb2_ref, m1t_ref, mb1_ref,
               m2t_ref, mb2_ref, f11at_ref, fb11_ref, f21at_ref, fb21_ref,
               feat_ref, v1_ref, u1_ref):
    z = q_ref[0]
    tb = _bdot(z, l2t_ref[...]) + lb2_ref[...]
    y = _bdot(tb, c2t_ref[...]) + cb2_ref[...]
    mx = jnp.max(y, axis=0, keepdims=True)
    t = _relu(_bdot(mx, m1t_ref[...]) + mb1_ref[...])
    feat = _bdot(t, m2t_ref[...]) + mb2_ref[...]
    feat_ref[0] = feat
    v1_ref[0] = _bdot(feat, f11at_ref[...]) + fb11_ref[...]
    u1_ref[0] = _bdot(feat, f21at_ref[...]) + fb21_ref[...]


def _head(q3, l2t, lb2, c2t, cb2, m1t, mb1, m2t, mb2, f11at, fb11,
          f21at, fb21):
    b, n, c = q3.shape
    full = lambda a: pl.BlockSpec(a.shape, lambda i: (0,) * a.ndim)
    return pl.pallas_call(
        _head_body,
        grid=(b,),
        in_specs=[
            pl.BlockSpec((1, n, c), lambda i: (i, 0, 0)),
            full(l2t), full(lb2[None]), full(c2t), full(cb2[None]),
            full(m1t), full(mb1[None]), full(m2t), full(mb2[None]),
            full(f11at), full(fb11[None]), full(f21at), full(fb21[None]),
        ],
        out_specs=[
            pl.BlockSpec((1, 1, 512), lambda i: (i, 0, 0)),
            pl.BlockSpec((1, 1, 512), lambda i: (i, 0, 0)),
            pl.BlockSpec((1, 1, 512), lambda i: (i, 0, 0)),
        ],
        out_shape=[
            jax.ShapeDtypeStruct((b, 1, 512), F32),
            jax.ShapeDtypeStruct((b, 1, 512), F32),
            jax.ShapeDtypeStruct((b, 1, 512), F32),
        ],
        interpret=_INTERPRET,
    )(q3, l2t, lb2[None], c2t, cb2[None], m1t, mb1[None], m2t, mb2[None],
      f11at, fb11[None], f21at, fb21[None])


def _fold_body(v1_ref, u1_ref, wg_ref, f12t_ref, fb12_ref, f13t_ref, fb13_ref,
               f21bt_ref, f22t_ref, fb22_ref, f23t_ref, fb23_ref, o_ref,
               *, blk):
    j = pl.program_id(1)
    step = jnp.float32(1.0) / jnp.float32(M_PTS - 1)
    gcol = (jax.lax.broadcasted_iota(jnp.int32, (blk, 1), 0)
            + j * blk).astype(F32) * step
    gb = gcol.astype(BF).astype(F32)
    f1a = _relu(v1_ref[0] + wg_ref[...] * gb)
    h1 = _relu(_bdot(f1a, f12t_ref[...]) + fb12_ref[...])
    f1 = _bdot(h1, f13t_ref[...]) + fb13_ref[...]
    g1a = _relu(u1_ref[0] + _bdot(f1, f21bt_ref[...]))
    h2 = _relu(_bdot(g1a, f22t_ref[...]) + fb22_ref[...])
    f2 = _bdot(h2, f23t_ref[...]) + fb23_ref[...]
    o_ref[0] = f2


def _fold(v1, u1, wg, f12t, fb12, f13t, fb13, f21bt, f22t, fb22, f23t, fb23):
    b = v1.shape[0]
    blk = 512
    full = lambda a: pl.BlockSpec(a.shape, lambda i, j: (0,) * a.ndim)
    row = lambda a: pl.BlockSpec((1, 1, a.shape[2]), lambda i, j: (i, 0, 0))
    return pl.pallas_call(
        functools.partial(_fold_body, blk=blk),
        grid=(b, M_PTS // blk),
        in_specs=[
            row(v1), row(u1), full(wg[None]),
            full(f12t), full(fb12[None]), full(f13t), full(fb13[None]),
            full(f21bt), full(f22t), full(fb22[None]),
            full(f23t), full(fb23[None]),
        ],
        out_specs=pl.BlockSpec((1, blk, 2), lambda i, j: (i, j, 0)),
        out_shape=jax.ShapeDtypeStruct((b, M_PTS, 2), F32),
        interpret=_INTERPRET,
    )(v1, u1, wg[None], f12t, fb12[None], f13t, fb13[None], f21bt,
      f22t, fb22[None], f23t, fb23[None])


def kernel(input, W1, b1, W2, b2, W3, b3, L1, lb1, C1, cb1, L2, lb2, C2, cb2,
           M1, mb1, M2, mb2, F11, fb11, F12, fb12, F13, fb13, F21, fb21,
           F22, fb22, F23, fb23):
    b, n, _ = input.shape
    tb = lambda w: w.T.astype(BF)

    wg = F11[:, 512].astype(BF).astype(F32)

    idx, h3 = _knn(input, tb(W1), b1, tb(W2), b2, tb(W3), b3)
    idx_flat = idx.reshape(b * n * K_NN)
    h = h3.reshape(b * n, 128)
    t1 = _lmax_sc(idx_flat, h, 64)
    q = _lin2(t1, tb(L1), lb1, tb(C1), cb1)
    t2 = _lmax_sc(idx_flat, q, 128)
    feat, v1, u1 = _head(t2.reshape(b, n, 128), tb(L2), lb2, tb(C2), cb2,
                         tb(M1), mb1, tb(M2), mb2,
                         tb(F11[:, :512]), fb11, tb(F21[:, :512]), fb21)
    out = _fold(v1, u1, wg, tb(F12), fb12, tb(F13), fb13,
                tb(F21[:, 512:514]), tb(F22), fb22, tb(F23), fb23)
    return out, feat
